# baseline (speedup 1.0000x reference)
import functools

import jax
import jax.numpy as jnp
from jax import lax
from jax.experimental import pallas as pl
from jax.experimental.pallas import tpu as pltpu
from jax.experimental.pallas import tpu_sc as plsc

BINS = 256
NUM_WORKERS = 32
NCH = 4
ROW = 147456
CH = ROW // NCH
HPC = 384 // NCH
ROWS = 384
SPLITS = 4
RH = ROWS // SPLITS
RPW = RH // NUM_WORKERS
PW = ((RPW * 4 + 16 + 15) // 16) * 16


def _minmax_body(s_ref, t_ref, mins_ref, maxs_ref, mint_ref, maxt_ref):
    xs = s_ref[...]
    xt = t_ref[...]
    mins_ref[...] = jnp.broadcast_to(jnp.min(xs, axis=(1, 2))[:, None], mins_ref.shape)
    maxs_ref[...] = jnp.broadcast_to(jnp.max(xs, axis=(1, 2))[:, None], maxs_ref.shape)
    mint_ref[...] = jnp.broadcast_to(jnp.min(xt, axis=(1, 2))[:, None], mint_ref.shape)
    maxt_ref[...] = jnp.broadcast_to(jnp.max(xt, axis=(1, 2))[:, None], maxt_ref.shape)


def _row_minmax(s3, t3, half):
    rb = 8
    grid = (RH // rb,)
    base = half * (RH // rb)
    out = jax.ShapeDtypeStruct((RH, 128), jnp.float32)
    return pl.pallas_call(
        _minmax_body,
        grid=grid,
        in_specs=[
            pl.BlockSpec((rb, 384, 384), lambda i: (i + base, 0, 0)),
            pl.BlockSpec((rb, 384, 384), lambda i: (i + base, 0, 0)),
        ],
        out_specs=[pl.BlockSpec((rb, 128), lambda i: (i, 0))] * 4,
        out_shape=[out] * 4,
    )(s3, t3)


def _make_sc_body(half):
    return functools.partial(_sc_hist_kernel, half * RH)


def _sc_hist_kernel(row_base, src_hbm, tgt_hbm, par_hbm, out_hbm, buf0, buf1,
                    hist, pv, ov, dacc, sem0, sem1):
    wid = lax.axis_index("s") * 2 + lax.axis_index("c")
    lane = lax.iota(jnp.int32, 16)
    lane_base_f = (lane * BINS).astype(jnp.float32)
    zeros16 = jnp.zeros((16,), jnp.float32)
    dacc[...] = zeros16

    pltpu.sync_copy(par_hbm.at[wid], pv)

    pltpu.async_copy(src_hbm.at[row_base + wid * RPW, pl.ds(0, HPC)], buf0, sem0)

    nvisit = RPW * 2 * NCH

    def process(buf_ref, sv, bv, vv):
        @plsc.parallel_loop(0, HPC, 1, unroll=1)
        def _(r):
            for g in range(384 // 16):
                x = buf_ref[r, pl.ds(g * 16, 16)]
                t = x * sv + bv
                idx = (t.astype(jnp.int32) << 4) | lane
                plsc.addupdate_scatter(hist, [idx], vv)

    def visit(v, _carry):
        j = v // (2 * NCH)
        tensor = (v // NCH) % 2
        c = v % NCH
        b = v % 2

        v2 = v + 1
        j2 = v2 // (2 * NCH)
        t2 = (v2 // NCH) % 2
        c2 = v2 % NCH
        b2 = v2 % 2
        row2 = row_base + wid * RPW + j2

        for tb, tref in ((0, src_hbm), (1, tgt_hbm)):
            for bb, bref, sref in ((0, buf0, sem0), (1, buf1, sem1)):
                @pl.when((v2 < nvisit) & (t2 == tb) & (b2 == bb))
                def _(tref=tref, bref=bref, sref=sref):
                    pltpu.async_copy(tref.at[row2, pl.ds(c2 * HPC, HPC)], bref, sref)

        @pl.when((tensor == 0) & (c == 0))
        def _():
            def zstep(k, _):
                base = k * (16 * 16)
                for u in range(16):
                    hist[pl.ds(base + u * 16, 16)] = zeros16
                return _
            lax.fori_loop(0, (16 * BINS) // (16 * 16), zstep, 0)
            hist[pl.ds(16 * BINS, 16)] = zeros16

        pj = pv[pl.ds(j * 4, 16)]
        s_s = pj[0]
        b_s = pj[1]
        s_t = pj[2]
        b_t = pj[3]
        is_src = tensor == 0
        sv = jnp.full((16,), jnp.where(is_src, s_s, s_t), jnp.float32)
        bv = jnp.full((16,), jnp.where(is_src, b_s, b_t), jnp.float32)
        vv = jnp.full((16,), jnp.where(is_src, 1.0, -1.0), jnp.float32)

        @pl.when(b == 0)
        def _():
            pltpu.make_async_copy(src_hbm.at[0, pl.ds(0, HPC)], buf0, sem0).wait()
            process(buf0, sv, bv, vv)

        @pl.when(b == 1)
        def _():
            pltpu.make_async_copy(src_hbm.at[0, pl.ds(0, HPC)], buf1, sem1).wait()
            process(buf1, sv, bv, vv)

        @pl.when((tensor == 1) & (c == NCH - 1))
        def _():
            @plsc.parallel_loop(0, BINS, 1, unroll=4, carry=jnp.float32(0.0))
            def dsum(b, acc):
                s = jnp.sum(hist[pl.ds(b * 16, 16)])
                return acc + s * s
            dacc[...] = dacc[...] + jnp.where(lane == 0, dsum, 0.0)

        return _carry

    lax.fori_loop(0, nvisit, visit, 0)

    total = jnp.sum(dacc[...]) * (1.0 / (BINS * BINS * BINS))
    ov[...] = jnp.where(lane == 0, total, 0.0)
    pltpu.sync_copy(ov, out_hbm.at[wid])


def _sc_hist(src3, tgt3, par3, half):
    mesh = plsc.VectorSubcoreMesh(core_axis_name="c", subcore_axis_name="s")
    kfn = functools.partial(
        pl.kernel,
        mesh=mesh,
        out_type=jax.ShapeDtypeStruct((NUM_WORKERS, 16), jnp.float32),
        scratch_types=[
            pltpu.VMEM((HPC, 384), jnp.float32),
            pltpu.VMEM((HPC, 384), jnp.float32),
            pltpu.VMEM((16 * BINS + 16,), jnp.float32),
            pltpu.VMEM((PW,), jnp.float32),
            pltpu.VMEM((16,), jnp.float32),
            pltpu.VMEM((16,), jnp.float32),
            pltpu.SemaphoreType.DMA,
            pltpu.SemaphoreType.DMA,
        ],
        compiler_params=pltpu.CompilerParams(needs_layout_passes=False),
    )(_make_sc_body(half))
    return kfn(src3, tgt3, par3)


def kernel(source_tensor, target_tensor):
    N, C, H, W = source_tensor.shape
    s3 = source_tensor.reshape(ROWS, H, W)
    t3 = target_tensor.reshape(ROWS, H, W)

    total = jnp.float32(0.0)
    for half in range(SPLITS):
        mins, maxs, mint, maxt = _row_minmax(s3, t3, half)
        lo_s, hi_s = mins[:, 0], maxs[:, 0]
        lo_t, hi_t = mint[:, 0], maxt[:, 0]
        w_s = jnp.where(hi_s > lo_s, hi_s - lo_s, 1.0)
        w_t = jnp.where(hi_t > lo_t, hi_t - lo_t, 1.0)
        s_num = BINS * (1.0 - 2.0**-20)
        sc_s = s_num / w_s
        sc_t = s_num / w_t
        par = jnp.stack([sc_s, -lo_s * sc_s, sc_t, -lo_t * sc_t], axis=1)
        par3 = jnp.pad(par.reshape(NUM_WORKERS, RPW * 4),
                       ((0, 0), (0, PW - RPW * 4)))
        total = total + jnp.sum(_sc_hist(s3, t3, par3, half))
    return total / (N * C)


# speedup vs baseline: 1.0031x; 1.0031x over previous
import functools

import jax
import jax.numpy as jnp
from jax import lax
from jax.experimental import pallas as pl
from jax.experimental.pallas import tpu as pltpu
from jax.experimental.pallas import tpu_sc as plsc

BINS = 256
NUM_WORKERS = 32
NCH = 4
ROW = 147456
CH = ROW // NCH
HPC = 384 // NCH
ROWS = 384
SPLITS = 2
RH = ROWS // SPLITS
RPW = RH // NUM_WORKERS
PW = ((RPW * 4 + 16 + 15) // 16) * 16


def _minmax_body(s_ref, t_ref, mins_ref, maxs_ref, mint_ref, maxt_ref):
    xs = s_ref[...]
    xt = t_ref[...]
    mins_ref[...] = jnp.broadcast_to(jnp.min(xs, axis=(1, 2))[:, None], mins_ref.shape)
    maxs_ref[...] = jnp.broadcast_to(jnp.max(xs, axis=(1, 2))[:, None], maxs_ref.shape)
    mint_ref[...] = jnp.broadcast_to(jnp.min(xt, axis=(1, 2))[:, None], mint_ref.shape)
    maxt_ref[...] = jnp.broadcast_to(jnp.max(xt, axis=(1, 2))[:, None], maxt_ref.shape)


def _row_minmax(s3, t3, half):
    rb = 8
    grid = (RH // rb,)
    base = half * (RH // rb)
    out = jax.ShapeDtypeStruct((RH, 128), jnp.float32)
    return pl.pallas_call(
        _minmax_body,
        grid=grid,
        in_specs=[
            pl.BlockSpec((rb, 384, 384), lambda i: (i + base, 0, 0)),
            pl.BlockSpec((rb, 384, 384), lambda i: (i + base, 0, 0)),
        ],
        out_specs=[pl.BlockSpec((rb, 128), lambda i: (i, 0))] * 4,
        out_shape=[out] * 4,
    )(s3, t3)


def _make_sc_body(half):
    return functools.partial(_sc_hist_kernel, half * RH)


def _sc_hist_kernel(row_base, src_hbm, tgt_hbm, par_hbm, out_hbm, buf0, buf1,
                    hist, pv, ov, dacc, sem0, sem1):
    wid = lax.axis_index("s") * 2 + lax.axis_index("c")
    lane = lax.iota(jnp.int32, 16)
    lane_base_f = (lane * BINS).astype(jnp.float32)
    zeros16 = jnp.zeros((16,), jnp.float32)
    dacc[...] = zeros16

    pltpu.sync_copy(par_hbm.at[wid], pv)

    pltpu.async_copy(src_hbm.at[row_base + wid * RPW, pl.ds(0, HPC)], buf0, sem0)

    nvisit = RPW * 2 * NCH

    def process(buf_ref, sv, bv, vv):
        @plsc.parallel_loop(0, HPC, 1, unroll=1)
        def _(r):
            for g in range(384 // 16):
                x = buf_ref[r, pl.ds(g * 16, 16)]
                t = x * sv + bv
                idx = (t.astype(jnp.int32) << 4) | lane
                plsc.addupdate_scatter(hist, [idx], vv)

    def visit(v, _carry):
        j = v // (2 * NCH)
        tensor = (v // NCH) % 2
        c = v % NCH
        b = v % 2

        v2 = v + 1
        j2 = v2 // (2 * NCH)
        t2 = (v2 // NCH) % 2
        c2 = v2 % NCH
        b2 = v2 % 2
        row2 = row_base + wid * RPW + j2

        for tb, tref in ((0, src_hbm), (1, tgt_hbm)):
            for bb, bref, sref in ((0, buf0, sem0), (1, buf1, sem1)):
                @pl.when((v2 < nvisit) & (t2 == tb) & (b2 == bb))
                def _(tref=tref, bref=bref, sref=sref):
                    pltpu.async_copy(tref.at[row2, pl.ds(c2 * HPC, HPC)], bref, sref)

        @pl.when((tensor == 0) & (c == 0))
        def _():
            def zstep(k, _):
                base = k * (16 * 16)
                for u in range(16):
                    hist[pl.ds(base + u * 16, 16)] = zeros16
                return _
            lax.fori_loop(0, (16 * BINS) // (16 * 16), zstep, 0)
            hist[pl.ds(16 * BINS, 16)] = zeros16

        pj = pv[pl.ds(j * 4, 16)]
        s_s = pj[0]
        b_s = pj[1]
        s_t = pj[2]
        b_t = pj[3]
        is_src = tensor == 0
        sv = jnp.full((16,), jnp.where(is_src, s_s, s_t), jnp.float32)
        bv = jnp.full((16,), jnp.where(is_src, b_s, b_t), jnp.float32)
        vv = jnp.full((16,), jnp.where(is_src, 1.0, -1.0), jnp.float32)

        @pl.when(b == 0)
        def _():
            pltpu.make_async_copy(src_hbm.at[0, pl.ds(0, HPC)], buf0, sem0).wait()
            process(buf0, sv, bv, vv)

        @pl.when(b == 1)
        def _():
            pltpu.make_async_copy(src_hbm.at[0, pl.ds(0, HPC)], buf1, sem1).wait()
            process(buf1, sv, bv, vv)

        @pl.when((tensor == 1) & (c == NCH - 1))
        def _():
            @plsc.parallel_loop(0, BINS, 1, unroll=4, carry=jnp.float32(0.0))
            def dsum(b, acc):
                s = jnp.sum(hist[pl.ds(b * 16, 16)])
                return acc + s * s
            dacc[...] = dacc[...] + jnp.where(lane == 0, dsum, 0.0)

        return _carry

    lax.fori_loop(0, nvisit, visit, 0)

    total = jnp.sum(dacc[...]) * (1.0 / (BINS * BINS * BINS))
    ov[...] = jnp.where(lane == 0, total, 0.0)
    pltpu.sync_copy(ov, out_hbm.at[wid])


def _sc_hist(src3, tgt3, par3, half):
    mesh = plsc.VectorSubcoreMesh(core_axis_name="c", subcore_axis_name="s")
    kfn = functools.partial(
        pl.kernel,
        mesh=mesh,
        out_type=jax.ShapeDtypeStruct((NUM_WORKERS, 16), jnp.float32),
        scratch_types=[
            pltpu.VMEM((HPC, 384), jnp.float32),
            pltpu.VMEM((HPC, 384), jnp.float32),
            pltpu.VMEM((16 * BINS + 16,), jnp.float32),
            pltpu.VMEM((PW,), jnp.float32),
            pltpu.VMEM((16,), jnp.float32),
            pltpu.VMEM((16,), jnp.float32),
            pltpu.SemaphoreType.DMA,
            pltpu.SemaphoreType.DMA,
        ],
        compiler_params=pltpu.CompilerParams(needs_layout_passes=False),
    )(_make_sc_body(half))
    return kfn(src3, tgt3, par3)


def kernel(source_tensor, target_tensor):
    N, C, H, W = source_tensor.shape
    s3 = source_tensor.reshape(ROWS, H, W)
    t3 = target_tensor.reshape(ROWS, H, W)

    total = jnp.float32(0.0)
    for half in range(SPLITS):
        mins, maxs, mint, maxt = _row_minmax(s3, t3, half)
        lo_s, hi_s = mins[:, 0], maxs[:, 0]
        lo_t, hi_t = mint[:, 0], maxt[:, 0]
        w_s = jnp.where(hi_s > lo_s, hi_s - lo_s, 1.0)
        w_t = jnp.where(hi_t > lo_t, hi_t - lo_t, 1.0)
        s_num = BINS * (1.0 - 2.0**-20)
        sc_s = s_num / w_s
        sc_t = s_num / w_t
        par = jnp.stack([sc_s, -lo_s * sc_s, sc_t, -lo_t * sc_t], axis=1)
        par3 = jnp.pad(par.reshape(NUM_WORKERS, RPW * 4),
                       ((0, 0), (0, PW - RPW * 4)))
        total = total + jnp.sum(_sc_hist(s3, t3, par3, half))
    return total / (N * C)
